# baseline (speedup 1.0000x reference)
import jax
import jax.numpy as jnp
from jax import lax
from jax.experimental import pallas as pl
from jax.experimental.pallas import tpu as pltpu
from jax.experimental.pallas import tpu_sc as plsc

VOCAB = 1000
B_DIM = 1024
NC, NS = 2, 16
NW = NC * NS
VCH = 40
KCH = VOCAB // VCH
LANES = 16
NGROUPS = B_DIM // LANES


def _sc_onehot_t(ids_hbm, zeros_hbm, out_hbm,
                 row0, row1, buf0, buf1, sem0, sem1):
    L = ids_hbm.shape[0]
    n_slabs = L * KCH
    wid = lax.axis_index("s") * NC + lax.axis_index("c")

    pltpu.sync_copy(zeros_hbm, buf0)
    pltpu.sync_copy(zeros_hbm, buf1)

    iota = lax.iota(jnp.int32, LANES)
    ones_v = jnp.ones((LANES,), jnp.float32)
    zeros_v = jnp.zeros((LANES,), jnp.float32)
    rows = (row0, row1)
    bufs = (buf0, buf1)
    sems = (sem0, sem1)

    def scatter(buf, row, v0, vals):
        @pl.loop(0, NGROUPS)
        def _g(i):
            idv = row[pl.ds(i * LANES, LANES)]
            m = (idv >= v0) & (idv < v0 + VCH)
            plsc.store_scatter(buf, [idv - v0, iota + i * LANES], vals, mask=m)

    def dma(b, s):
        l = s // KCH
        v0 = (s % KCH) * VCH
        return pltpu.make_async_copy(
            bufs[b], out_hbm.at[l, pl.ds(v0, VCH), :], sems[b])

    def load_row(b, s):
        pltpu.sync_copy(ids_hbm.at[s // KCH], rows[b])

    def set_and_send(b, s):
        load_row(b, s)
        scatter(bufs[b], rows[b], (s % KCH) * VCH, ones_v)
        dma(b, s).start()

    for b in range(2):
        s = wid + b * NW

        @pl.when(s < n_slabs)
        def _():
            set_and_send(b, s)

    @pl.loop(2, (n_slabs - 1) // NW + 1)
    def _steady(t):
        s = wid + t * NW
        b = t % 2

        @pl.when(s < n_slabs)
        def _():
            for bb in range(2):
                @pl.when(b == bb)
                def _():
                    dma(bb, s - 2 * NW).wait()
                    scatter(bufs[bb], rows[bb],
                            ((s - 2 * NW) % KCH) * VCH, zeros_v)
                    set_and_send(bb, s)

    for b in range(2):
        s = wid + b * NW

        @pl.when(s < n_slabs)
        def _():
            dma(b, s).wait()


def kernel(input_ids) -> jnp.ndarray:
    B, L = input_ids.shape
    ids_t = input_ids.T.astype(jnp.int32)
    zeros_blk = jnp.zeros((VCH, B_DIM), jnp.float32)

    run = pl.kernel(
        _sc_onehot_t,
        out_type=jax.ShapeDtypeStruct((L, VOCAB, B_DIM), jnp.float32),
        mesh=plsc.VectorSubcoreMesh(
            core_axis_name="c", subcore_axis_name="s",
            num_cores=NC, num_subcores=NS),
        compiler_params=pltpu.CompilerParams(
            needs_layout_passes=False, use_tc_tiling_on_sc=True),
        scratch_types=[
            pltpu.VMEM((B_DIM,), jnp.int32),
            pltpu.VMEM((B_DIM,), jnp.int32),
            pltpu.VMEM((VCH, B_DIM), jnp.float32),
            pltpu.VMEM((VCH, B_DIM), jnp.float32),
            pltpu.SemaphoreType.DMA,
            pltpu.SemaphoreType.DMA,
        ],
    )
    out = run(ids_t, zeros_blk)
    return out.transpose(2, 0, 1)


# speedup vs baseline: 2.0251x; 2.0251x over previous
import jax
import jax.numpy as jnp
from jax.experimental import pallas as pl

VOCAB = 1000


def _onehot_block(ids_ref, out_ref):
    ids = ids_ref[pl.program_id(0), :]
    iota = jax.lax.broadcasted_iota(jnp.int32, (VOCAB, ids.shape[0]), 0)
    out_ref[0, :, :] = jnp.where(iota == ids[None, :], 1.0, 0.0)


def kernel(input_ids) -> jnp.ndarray:
    B, L = input_ids.shape
    ids_t = input_ids.T.astype(jnp.int32)
    out = pl.pallas_call(
        _onehot_block,
        grid=(L,),
        in_specs=[pl.BlockSpec((L, B), lambda i: (0, 0))],
        out_specs=pl.BlockSpec((1, VOCAB, B), lambda i: (i, 0, 0)),
        out_shape=jax.ShapeDtypeStruct((L, VOCAB, B), jnp.float32),
    )(ids_t)
    return out.transpose(2, 0, 1)
